# baseline (speedup 1.0000x reference)
import jax
import jax.numpy as jnp
from jax.experimental import pallas as pl

N_LAYERS_TOTAL = 13
N_GRAPHS_K = 16


def _layer(x, src, dst, et, W, q, k, b):
    n = x.shape[0]
    xw = jnp.einsum('ni,rio->rno', x, W)
    qn = jnp.einsum('rno,o->rn', xw, q[:, 0])
    kn = jnp.einsum('rno,o->rn', xw, k[:, 0])
    qi = qn[et, dst][:, None]
    kj = kn[et, src][:, None]
    alpha = jax.nn.leaky_relu(qi + kj, negative_slope=0.2)
    amax = jax.ops.segment_max(alpha, dst, num_segments=n)
    amax = jnp.where(jnp.isfinite(amax), amax, 0.0)
    ex = jnp.exp(alpha - amax[dst])
    denom = jax.ops.segment_sum(ex, dst, num_segments=n)
    a = ex / (denom[dst] + 1e-16)
    out_j = xw[et, src]
    msg = a * out_j
    return jax.ops.segment_sum(msg, dst, num_segments=n) + b


def kernel(node_features, edge_index, edge_type, batch_index, W0, Ws, q_att, k_att, biases, t):
    src = edge_index[0]
    dst = edge_index[1]
    results = []
    h = _layer(node_features, src, dst, edge_type, W0, q_att[0], k_att[0], biases[0])
    results.append(h)
    for i in range(N_LAYERS_TOTAL - 1):
        h = _layer(h, src, dst, edge_type, Ws[i], q_att[i + 1], k_att[i + 1], biases[i + 1])
        results.append(h)
    node_representations = jnp.concatenate(results, axis=-1)
    alpha = node_representations * t
    amax = jax.ops.segment_max(alpha, batch_index, num_segments=N_GRAPHS_K)
    amax = jnp.where(jnp.isfinite(amax), amax, 0.0)
    ex = jnp.exp(alpha - amax[batch_index])
    denom = jax.ops.segment_sum(ex, batch_index, num_segments=N_GRAPHS_K)
    w = ex / (denom[batch_index] + 1e-16)
    graph_representations = jax.ops.segment_sum(node_representations * w, batch_index, num_segments=N_GRAPHS_K)
    return (graph_representations, node_representations)


# speedup vs baseline: 27.3766x; 27.3766x over previous
import functools

import jax
import jax.numpy as jnp
from jax import lax
from jax.experimental import pallas as pl
from jax.experimental.pallas import tpu as pltpu
from jax.experimental.pallas import tpu_sc as plsc

N = 10000
E = 320000
D_H = 64
R = 3
G = 16
NLAYERS = 13
NW = 32
NPS = 320
NT = NW * NPS
BN = 512
WA = 1024
WB = 128
EPAD = E + 2 * WA
NEG = -1e30

_f32 = jnp.float32
_i32 = jnp.int32


def _tc_layer(h, W, q, k, bias):
    in_d = W.shape[1]
    grid = (NT // BN,)
    q2 = q.reshape(1, D_H)
    k2 = k.reshape(1, D_H)
    have_bias = bias is not None

    def body(*refs):
        if have_bias:
            h_ref, w_ref, q_ref, k_ref, b_ref, xw_ref, qn_ref, kn_ref, ho_ref = refs
        else:
            h_ref, w_ref, q_ref, k_ref, xw_ref, qn_ref, kn_ref = refs
        hb = h_ref[...]
        if have_bias:
            hb = hb + b_ref[0]
            ho_ref[...] = hb
        qv = q_ref[0]
        kv = k_ref[0]
        for r in range(R):
            xwr = jnp.dot(hb, w_ref[r], preferred_element_type=_f32,
                          precision=lax.Precision.HIGHEST)
            xw_ref[r] = xwr
            qn_ref[r] = jnp.dot(xwr, qv, preferred_element_type=_f32,
                                precision=lax.Precision.HIGHEST)
            kn_ref[r] = jnp.dot(xwr, kv, preferred_element_type=_f32,
                                precision=lax.Precision.HIGHEST)

    in_specs = [
        pl.BlockSpec((BN, in_d), lambda i: (i, 0)),
        pl.BlockSpec((R, in_d, D_H), lambda i: (0, 0, 0)),
        pl.BlockSpec((1, D_H), lambda i: (0, 0)),
        pl.BlockSpec((1, D_H), lambda i: (0, 0)),
    ]
    args = [h, W, q2, k2]
    if have_bias:
        in_specs.append(pl.BlockSpec((1, D_H), lambda i: (0, 0)))
        args.append(bias.reshape(1, D_H))
    out_shape = [
        jax.ShapeDtypeStruct((R, NT, D_H), _f32),
        jax.ShapeDtypeStruct((R, NT), _f32),
        jax.ShapeDtypeStruct((R, NT), _f32),
    ]
    out_specs = [
        pl.BlockSpec((R, BN, D_H), lambda i: (0, i, 0)),
        pl.BlockSpec((R, BN), lambda i: (0, i)),
        pl.BlockSpec((R, BN), lambda i: (0, i)),
    ]
    if have_bias:
        out_shape.append(jax.ShapeDtypeStruct((NT, D_H), _f32))
        out_specs.append(pl.BlockSpec((BN, D_H), lambda i: (i, 0)))
    outs = pl.pallas_call(
        body, grid=grid, in_specs=in_specs, out_specs=out_specs,
        out_shape=out_shape)(*args)
    if have_bias:
        return outs[0], outs[1], outs[2], outs[3]
    return outs[0], outs[1], outs[2], None


def _tc_bias(msg, bias):
    def body(m_ref, b_ref, o_ref):
        o_ref[...] = m_ref[...] + b_ref[0]

    return pl.pallas_call(
        body, grid=(NT // BN,),
        in_specs=[pl.BlockSpec((BN, D_H), lambda i: (i, 0)),
                  pl.BlockSpec((1, D_H), lambda i: (0, 0))],
        out_specs=pl.BlockSpec((BN, D_H), lambda i: (i, 0)),
        out_shape=jax.ShapeDtypeStruct((NT, D_H), _f32),
    )(msg, bias.reshape(1, D_H))


def _seg_scan(v, key, kbuf, vbuf, op):
    kbuf[pl.ds(16, 16)] = key
    for s in (1, 2, 4, 8):
        vbuf[pl.ds(16, 16)] = v
        sv = vbuf[pl.ds(16 - s, 16)]
        sk = kbuf[pl.ds(16 - s, 16)]
        v = jnp.where(sk == key, op(v, sv), v)
    nxt = kbuf[pl.ds(17, 16)]
    return v, key != nxt


def _sc_layer(xw_flat, qn_flat, kn_flat, qidx, kidx, dsts, ebnd):
    mesh = plsc.VectorSubcoreMesh(core_axis_name="c", subcore_axis_name="s")

    @functools.partial(
        pl.kernel, mesh=mesh,
        compiler_params=pltpu.CompilerParams(needs_layout_passes=False,
                                             use_tc_tiling_on_sc=False),
        out_type=jax.ShapeDtypeStruct((NT * D_H,), _f32),
        scratch_types=[
            pltpu.VMEM((R * NT,), _f32),
            pltpu.VMEM((R * NT,), _f32),
            pltpu.VMEM((64,), _i32),
            pltpu.VMEM((NPS,), _f32),
            pltpu.VMEM((NPS,), _f32),
            pltpu.VMEM((NPS * D_H,), _f32),
            pltpu.VMEM((WA,), _i32),
            pltpu.VMEM((WA,), _i32),
            pltpu.VMEM((WA,), _i32),
            pltpu.VMEM((48,), _f32),
            pltpu.VMEM((48,), _i32),
            pltpu.VMEM((WB,), _f32),
            pltpu.VMEM((WB,), _i32),
            pltpu.VMEM((WB, D_H), _f32),
            pltpu.SemaphoreType.DMA,
        ],
    )
    def sck(xw_h, qn_h, kn_h, qidx_h, kidx_h, dst_h, ebnd_h, msg_h,
            qn_v, kn_v, ebnd_v, amax_v, den_v, out_v,
            qw_v, kw_v, dw_v, vbuf, kbuf, a_v, ib_v, rb_v, sem):
        wid = lax.axis_index("c") * 16 + lax.axis_index("s")
        node_lo = wid * NPS

        pltpu.sync_copy(qn_h, qn_v)
        pltpu.sync_copy(kn_h, kn_v)
        pltpu.sync_copy(ebnd_h, ebnd_v)
        eb = ebnd_v[pl.ds(wid, 16)]
        e_lo = eb[0]
        e_hi = eb[1]
        e0 = e_lo - lax.rem(e_lo, 8)

        zeros16 = jnp.zeros((16,), _f32)
        neg16 = jnp.full((16,), NEG, _f32)

        def init_small(i, _):
            amax_v[pl.ds(i * 16, 16)] = neg16
            den_v[pl.ds(i * 16, 16)] = zeros16
            return 0
        lax.fori_loop(0, NPS // 16, init_small, 0)

        def init_out(i, _):
            out_v[pl.ds(i * 16, 16)] = zeros16
            return 0
        lax.fori_loop(0, NPS * D_H // 16, init_out, 0)

        kbuf[pl.ds(0, 16)] = jnp.full((16,), -1, _i32)
        kbuf[pl.ds(32, 16)] = jnp.full((16,), -2, _i32)

        iota16 = lax.iota(_i32, 16)
        nwin_a = (e_hi - e0 + (WA - 1)) // WA

        def alpha_of(base, estart):
            qi = plsc.load_gather(qn_v, [qw_v[pl.ds(base, 16)]])
            kj = plsc.load_gather(kn_v, [kw_v[pl.ds(base, 16)]])
            s = qi + kj
            alpha = jnp.where(s >= 0.0, s, s * jnp.float32(0.2))
            ev = estart + base + iota16
            valid = (ev >= e_lo) & (ev < e_hi)
            key = dw_v[pl.ds(base, 16)]
            kloc = jnp.clip(key - node_lo, 0, NPS - 1)
            return alpha, valid, key, kloc

        def stage_a(estart):
            estart = pl.multiple_of(estart, 8)
            pltpu.sync_copy(qidx_h.at[pl.ds(estart, WA)], qw_v)
            pltpu.sync_copy(kidx_h.at[pl.ds(estart, WA)], kw_v)
            pltpu.sync_copy(dst_h.at[pl.ds(estart, WA)], dw_v)

        vbuf[pl.ds(0, 16)] = neg16

        def pass1_win(j, _):
            estart = e0 + j * WA
            stage_a(estart)

            def vec(v, _):
                base = v * 16
                alpha, valid, key, kloc = alpha_of(base, estart)
                alpha = jnp.where(valid, alpha, jnp.float32(NEG))
                sm, last = _seg_scan(alpha, key, kbuf, vbuf, jnp.maximum)
                cur = plsc.load_gather(amax_v, [kloc])
                plsc.store_scatter(amax_v, [kloc], jnp.maximum(cur, sm),
                                   mask=last & valid)
                return 0
            lax.fori_loop(0, WA // 16, vec, 0)
            return 0
        lax.fori_loop(0, nwin_a, pass1_win, 0)

        vbuf[pl.ds(0, 16)] = zeros16

        def pass2_win(j, _):
            estart = e0 + j * WA
            stage_a(estart)

            def vec(v, _):
                base = v * 16
                alpha, valid, key, kloc = alpha_of(base, estart)
                am = plsc.load_gather(amax_v, [kloc])
                ex = jnp.exp(jnp.where(valid, alpha - am, jnp.float32(NEG)))
                ss, last = _seg_scan(ex, key, kbuf, vbuf, lambda a, b: a + b)
                cur = plsc.load_gather(den_v, [kloc])
                plsc.store_scatter(den_v, [kloc], cur + ss,
                                   mask=last & valid)
                return 0
            lax.fori_loop(0, WA // 16, vec, 0)
            return 0
        lax.fori_loop(0, nwin_a, pass2_win, 0)

        nwin_b = (e_hi - e0 + (WB - 1)) // WB

        def passb_win(j, _):
            estart = pl.multiple_of(e0 + j * WB, 8)
            pltpu.sync_copy(qidx_h.at[pl.ds(estart, WB)],
                            qw_v.at[pl.ds(0, WB)])
            pltpu.sync_copy(kidx_h.at[pl.ds(estart, WB)], ib_v)
            pltpu.sync_copy(kidx_h.at[pl.ds(estart, WB)],
                            kw_v.at[pl.ds(0, WB)])
            pltpu.sync_copy(dst_h.at[pl.ds(estart, WB)],
                            dw_v.at[pl.ds(0, WB)])
            cp = pltpu.async_copy(xw_h.at[ib_v], rb_v, sem)

            def vec(v, _):
                base = v * 16
                alpha, valid, key, kloc = alpha_of(base, estart)
                am = plsc.load_gather(amax_v, [kloc])
                den = plsc.load_gather(den_v, [kloc])
                ex = jnp.exp(jnp.where(valid, alpha - am, jnp.float32(NEG)))
                a = ex / (den + jnp.float32(1e-16))
                a_v[pl.ds(base, 16)] = jnp.where(valid, a, jnp.float32(0.0))
                return 0
            lax.fori_loop(0, WB // 16, vec, 0)
            cp.wait()

            def edge16(v, _):
                base = v * 16
                a16 = a_v[pl.ds(base, 16)]
                d16 = dw_v[pl.ds(base, 16)]
                for lane in range(16):
                    ai = a16[lane]
                    dloc = jnp.clip(d16[lane] - node_lo, 0, NPS - 1)
                    off = dloc * D_H
                    for c in range(D_H // 16):
                        cur = out_v[pl.ds(off + c * 16, 16)]
                        out_v[pl.ds(off + c * 16, 16)] = (
                            cur + ai * rb_v[base + lane, pl.ds(c * 16, 16)])
                return 0
            lax.fori_loop(0, WB // 16, edge16, 0)
            return 0
        lax.fori_loop(0, nwin_b, passb_win, 0)

        pltpu.sync_copy(
            out_v, msg_h.at[pl.ds(pl.multiple_of(node_lo * D_H, 8),
                                  NPS * D_H)])

    return sck(xw_flat, qn_flat, kn_flat, qidx, kidx, dsts, ebnd)


def _tc_aggregate(nr, bix, t):
    DT = NLAYERS * D_H
    t2 = jnp.reshape(t, (1, 1))

    def body1(nr_ref, b_ref, t_ref, am_ref):
        @pl.when(pl.program_id(0) == 0)
        def _():
            am_ref[...] = jnp.full((G, DT), NEG, _f32)
        xb = nr_ref[...] * t_ref[0, 0]
        b = b_ref[...]
        for g in range(G):
            m = jnp.max(jnp.where(b == g, xb, jnp.float32(NEG)),
                        axis=0, keepdims=True)
            am_ref[pl.ds(g, 1), :] = jnp.maximum(am_ref[pl.ds(g, 1), :], m)

    amax = pl.pallas_call(
        body1, grid=(NT // BN,),
        in_specs=[pl.BlockSpec((BN, DT), lambda i: (i, 0)),
                  pl.BlockSpec((BN, 1), lambda i: (i, 0)),
                  pl.BlockSpec((1, 1), lambda i: (0, 0))],
        out_specs=pl.BlockSpec((G, DT), lambda i: (0, 0)),
        out_shape=jax.ShapeDtypeStruct((G, DT), _f32),
    )(nr, bix, t2)

    def body2(nr_ref, b_ref, t_ref, am_ref, o_ref, sex_ref, sxex_ref):
        i = pl.program_id(0)

        @pl.when(i == 0)
        def _():
            sex_ref[...] = jnp.zeros((G, DT), _f32)
            sxex_ref[...] = jnp.zeros((G, DT), _f32)
        x = nr_ref[...]
        xb = x * t_ref[0, 0]
        b = b_ref[...]
        oh = (b == lax.broadcasted_iota(_i32, (BN, G), 1)).astype(_f32)
        am_rows = jnp.dot(oh, am_ref[...], preferred_element_type=_f32,
                          precision=lax.Precision.HIGHEST)
        ex = jnp.exp(xb - am_rows)
        dn = (((0,), (0,)), ((), ()))
        sex_ref[...] += lax.dot_general(
            oh, ex, dn, preferred_element_type=_f32,
            precision=lax.Precision.HIGHEST)
        sxex_ref[...] += lax.dot_general(
            oh, x * ex, dn, preferred_element_type=_f32,
            precision=lax.Precision.HIGHEST)

        @pl.when(i == NT // BN - 1)
        def _():
            o_ref[...] = sxex_ref[...] / (sex_ref[...] + jnp.float32(1e-16))

    return pl.pallas_call(
        body2, grid=(NT // BN,),
        in_specs=[pl.BlockSpec((BN, DT), lambda i: (i, 0)),
                  pl.BlockSpec((BN, 1), lambda i: (i, 0)),
                  pl.BlockSpec((1, 1), lambda i: (0, 0)),
                  pl.BlockSpec((G, DT), lambda i: (0, 0))],
        out_specs=pl.BlockSpec((G, DT), lambda i: (0, 0)),
        out_shape=jax.ShapeDtypeStruct((G, DT), _f32),
        scratch_shapes=[pltpu.VMEM((G, DT), _f32),
                        pltpu.VMEM((G, DT), _f32)],
    )(nr, bix, t2, amax)


def kernel(node_features, edge_index, edge_type, batch_index,
           W0, Ws, q_att, k_att, biases, t):
    src = edge_index[0]
    dst = edge_index[1]
    perm = jnp.argsort(dst)
    dst_s = dst[perm]
    src_s = src[perm]
    et_s = edge_type[perm]
    qidx = et_s * NT + dst_s
    kidx = et_s * NT + src_s
    padi = jnp.zeros((EPAD - E,), _i32)
    qidx = jnp.concatenate([qidx, padi])
    kidx = jnp.concatenate([kidx, padi])
    dst_p = jnp.concatenate([dst_s, jnp.full((EPAD - E,), 1 << 28, _i32)])
    ebnd = jnp.searchsorted(
        dst_s, jnp.minimum(jnp.arange(33, dtype=_i32) * NPS, N)).astype(_i32)
    ebnd = jnp.concatenate([ebnd, jnp.full((31,), E, _i32)])

    x0 = jnp.concatenate(
        [node_features, jnp.zeros((NT - N, node_features.shape[1]), _f32)])
    bix = jnp.concatenate(
        [batch_index, jnp.full((NT - N,), G, _i32)]).reshape(NT, 1)

    results = []
    xw, qn, kn, _ = _tc_layer(x0, W0, q_att[0], k_att[0], None)
    msg = _sc_layer(xw.reshape(R * NT, D_H), qn.reshape(-1), kn.reshape(-1),
                    qidx, kidx, dst_p, ebnd).reshape(NT, D_H)
    for i in range(NLAYERS - 1):
        xw, qn, kn, h = _tc_layer(msg, Ws[i], q_att[i + 1], k_att[i + 1],
                                  biases[i])
        results.append(h)
        msg = _sc_layer(xw.reshape(R * NT, D_H), qn.reshape(-1),
                        kn.reshape(-1), qidx, kidx, dst_p, ebnd
                        ).reshape(NT, D_H)
    results.append(_tc_bias(msg, biases[NLAYERS - 1]))
    nr_pad = jnp.concatenate(results, axis=-1)
    graph_representations = _tc_aggregate(nr_pad, bix, t)
    node_representations = nr_pad[:N]
    return (graph_representations, node_representations)


# speedup vs baseline: 35.6986x; 1.3040x over previous
import functools

import jax
import jax.numpy as jnp
from jax import lax
from jax.experimental import pallas as pl
from jax.experimental.pallas import tpu as pltpu
from jax.experimental.pallas import tpu_sc as plsc

N = 10000
E = 320000
D_H = 64
R = 3
G = 16
NLAYERS = 13
NW = 32
NPS = 320
NT = NW * NPS
BN = 512
WA = 2048
SB = 256
NSB = WA // SB
DUMP = NPS
OUTR = 336
EPAD = E + 2 * WA
NEG = -1e30

_f32 = jnp.float32
_i32 = jnp.int32


def _tc_layer(h, W, q, k, bias):
    in_d = W.shape[1]
    grid = (NT // BN,)
    q2 = q.reshape(1, D_H)
    k2 = k.reshape(1, D_H)
    have_bias = bias is not None

    def body(*refs):
        if have_bias:
            h_ref, w_ref, q_ref, k_ref, b_ref, xw_ref, qn_ref, kn_ref, ho_ref = refs
        else:
            h_ref, w_ref, q_ref, k_ref, xw_ref, qn_ref, kn_ref = refs
        hb = h_ref[...]
        if have_bias:
            hb = hb + b_ref[0]
            ho_ref[...] = hb
        qv = q_ref[0]
        kv = k_ref[0]
        for r in range(R):
            xwr = jnp.dot(hb, w_ref[r], preferred_element_type=_f32,
                          precision=lax.Precision.HIGHEST)
            xw_ref[r] = xwr
            qn_ref[r] = jnp.dot(xwr, qv, preferred_element_type=_f32,
                                precision=lax.Precision.HIGHEST)
            kn_ref[r] = jnp.dot(xwr, kv, preferred_element_type=_f32,
                                precision=lax.Precision.HIGHEST)

    in_specs = [
        pl.BlockSpec((BN, in_d), lambda i: (i, 0)),
        pl.BlockSpec((R, in_d, D_H), lambda i: (0, 0, 0)),
        pl.BlockSpec((1, D_H), lambda i: (0, 0)),
        pl.BlockSpec((1, D_H), lambda i: (0, 0)),
    ]
    args = [h, W, q2, k2]
    if have_bias:
        in_specs.append(pl.BlockSpec((1, D_H), lambda i: (0, 0)))
        args.append(bias.reshape(1, D_H))
    out_shape = [
        jax.ShapeDtypeStruct((R, NT, D_H), _f32),
        jax.ShapeDtypeStruct((R, NT), _f32),
        jax.ShapeDtypeStruct((R, NT), _f32),
    ]
    out_specs = [
        pl.BlockSpec((R, BN, D_H), lambda i: (0, i, 0)),
        pl.BlockSpec((R, BN), lambda i: (0, i)),
        pl.BlockSpec((R, BN), lambda i: (0, i)),
    ]
    if have_bias:
        out_shape.append(jax.ShapeDtypeStruct((NT, D_H), _f32))
        out_specs.append(pl.BlockSpec((BN, D_H), lambda i: (i, 0)))
    outs = pl.pallas_call(
        body, grid=grid, in_specs=in_specs, out_specs=out_specs,
        out_shape=out_shape)(*args)
    if have_bias:
        return outs[0], outs[1], outs[2], outs[3]
    return outs[0], outs[1], outs[2], None


def _tc_bias(msg, bias):
    def body(m_ref, b_ref, o_ref):
        o_ref[...] = m_ref[...] + b_ref[0]

    return pl.pallas_call(
        body, grid=(NT // BN,),
        in_specs=[pl.BlockSpec((BN, D_H), lambda i: (i, 0)),
                  pl.BlockSpec((1, D_H), lambda i: (0, 0))],
        out_specs=pl.BlockSpec((BN, D_H), lambda i: (i, 0)),
        out_shape=jax.ShapeDtypeStruct((NT, D_H), _f32),
    )(msg, bias.reshape(1, D_H))


def _seg_scan(v, key, kbuf, vbuf, op):
    kbuf[pl.ds(16, 16)] = key
    for s in (1, 2, 4, 8):
        vbuf[pl.ds(16, 16)] = v
        sv = vbuf[pl.ds(16 - s, 16)]
        sk = kbuf[pl.ds(16 - s, 16)]
        v = jnp.where(sk == key, op(v, sv), v)
    nxt = kbuf[pl.ds(17, 16)]
    return v, key != nxt


def _sc_layer(xw_flat, qn_flat, kn_flat, e3, ebnd):
    mesh = plsc.VectorSubcoreMesh(core_axis_name="c", subcore_axis_name="s")

    @functools.partial(
        pl.kernel, mesh=mesh,
        compiler_params=pltpu.CompilerParams(needs_layout_passes=False,
                                             use_tc_tiling_on_sc=False),
        out_type=jax.ShapeDtypeStruct((NT * D_H,), _f32),
        scratch_types=[
            pltpu.VMEM((R * NT,), _f32),
            pltpu.VMEM((R * NT,), _f32),
            pltpu.VMEM((64,), _i32),
            pltpu.VMEM((NPS,), _f32),
            pltpu.VMEM((NPS,), _f32),
            pltpu.VMEM((OUTR * D_H,), _f32),
            pltpu.VMEM((R, WA), _i32),
            pltpu.VMEM((48,), _f32),
            pltpu.VMEM((48,), _i32),
            pltpu.VMEM((SB,), _f32),
            pltpu.VMEM((2, SB, D_H), _f32),
            pltpu.SemaphoreType.DMA,
            pltpu.SemaphoreType.DMA,
        ],
    )
    def sck(xw_h, qn_h, kn_h, e3_h, ebnd_h, msg_h,
            qn_v, kn_v, ebnd_v, amax_v, den_v, out_v,
            s3_v, vbuf, kbuf, a_v, rb_v, sem0, sem1):
        wid = lax.axis_index("c") * 16 + lax.axis_index("s")
        node_lo = wid * NPS

        pltpu.sync_copy(qn_h, qn_v)
        pltpu.sync_copy(kn_h, kn_v)
        pltpu.sync_copy(ebnd_h, ebnd_v)
        eb = ebnd_v[pl.ds(wid, 16)]
        e_lo = eb[0]
        e_hi = eb[1]
        e0 = e_lo - lax.rem(e_lo, 8)

        zeros16 = jnp.zeros((16,), _f32)
        neg16 = jnp.full((16,), NEG, _f32)

        def init_small(i, _):
            amax_v[pl.ds(i * 16, 16)] = neg16
            den_v[pl.ds(i * 16, 16)] = zeros16
            return 0
        lax.fori_loop(0, NPS // 16, init_small, 0)

        def init_out(i, _):
            out_v[pl.ds(i * 16, 16)] = zeros16
            return 0
        lax.fori_loop(0, OUTR * D_H // 16, init_out, 0)

        kbuf[pl.ds(0, 16)] = jnp.full((16,), -1, _i32)
        kbuf[pl.ds(32, 16)] = jnp.full((16,), -2, _i32)

        iota16 = lax.iota(_i32, 16)
        nwin = (e_hi - e0 + (WA - 1)) // WA

        def stage(estart):
            estart = pl.multiple_of(estart, 8)
            pltpu.sync_copy(e3_h.at[:, pl.ds(estart, WA)], s3_v)

        def alpha_of(base, estart):
            qi = plsc.load_gather(qn_v, [s3_v[0, pl.ds(base, 16)]])
            kj = plsc.load_gather(kn_v, [s3_v[1, pl.ds(base, 16)]])
            s = qi + kj
            alpha = jnp.where(s >= 0.0, s, s * jnp.float32(0.2))
            ev = estart + base + iota16
            valid = (ev >= e_lo) & (ev < e_hi)
            key = s3_v[2, pl.ds(base, 16)]
            kloc = jnp.clip(key - node_lo, 0, NPS - 1)
            return alpha, valid, key, kloc

        vbuf[pl.ds(0, 16)] = neg16

        def pass1_win(j, _):
            estart = e0 + j * WA
            stage(estart)

            def vec(v, _):
                base = v * 16
                alpha, valid, key, kloc = alpha_of(base, estart)
                alpha = jnp.where(valid, alpha, jnp.float32(NEG))
                sm, last = _seg_scan(alpha, key, kbuf, vbuf, jnp.maximum)
                cur = plsc.load_gather(amax_v, [kloc])
                plsc.store_scatter(amax_v, [kloc], jnp.maximum(cur, sm),
                                   mask=last & valid)
                return 0
            lax.fori_loop(0, WA // 16, vec, 0)
            return 0
        lax.fori_loop(0, nwin, pass1_win, 0)

        vbuf[pl.ds(0, 16)] = zeros16

        def pass2_win(j, _):
            estart = e0 + j * WA
            stage(estart)

            def vec(v, _):
                base = v * 16
                alpha, valid, key, kloc = alpha_of(base, estart)
                am = plsc.load_gather(amax_v, [kloc])
                ex = jnp.exp(jnp.where(valid, alpha - am, jnp.float32(NEG)))
                ss, last = _seg_scan(ex, key, kbuf, vbuf, lambda a, b: a + b)
                cur = plsc.load_gather(den_v, [kloc])
                plsc.store_scatter(den_v, [kloc], cur + ss,
                                   mask=last & valid)
                return 0
            lax.fori_loop(0, WA // 16, vec, 0)
            return 0
        lax.fori_loop(0, nwin, pass2_win, 0)

        sems = (sem0, sem1)

        def fire(s):
            par = s % 2
            sbase = s * SB
            cps = []
            for h in range(SB // 128):
                idx_ref = s3_v.at[1, pl.ds(sbase + h * 128, 128)]
                dst_ref = rb_v.at[par, pl.ds(h * 128, 128), :]
                cps.append(pltpu.async_copy(xw_h.at[idx_ref], dst_ref,
                                            sems[par]))
            return cps

        def passb_win(j, carry):
            estart = e0 + j * WA
            stage(estart)
            carry_in = carry
            cps = fire(0)
            for s in range(NSB):
                par = s % 2
                sbase = s * SB
                nxt = fire(s + 1) if s + 1 < NSB else None
                def avec(v, _):
                    base = sbase + v * 16
                    alpha, valid, key, kloc = alpha_of(base, estart)
                    am = plsc.load_gather(amax_v, [kloc])
                    den = plsc.load_gather(den_v, [kloc])
                    ex = jnp.exp(jnp.where(valid, alpha - am,
                                           jnp.float32(NEG)))
                    a = ex / (den + jnp.float32(1e-16))
                    a_v[pl.ds(v * 16, 16)] = jnp.where(valid, a,
                                                       jnp.float32(0.0))
                    return 0
                lax.fori_loop(0, SB // 16, avec, 0)
                for cp in cps:
                    cp.wait()

                def edgegrp(g, car):
                    acc0, acc1, acc2, acc3, prev = car
                    accs = [acc0, acc1, acc2, acc3]
                    a16 = a_v[pl.ds(g * 16, 16)]
                    d16 = s3_v[2, pl.ds(sbase + g * 16, 16)]
                    for lane in range(16):
                        ai = a16[lane]
                        di = d16[lane]
                        dl = di - node_lo
                        ok = (dl >= 0) & (dl < NPS)
                        dloc = jnp.where(ok, dl, DUMP)
                        same = di == prev
                        off = dloc * D_H
                        for c in range(D_H // 16):
                            acc = jnp.where(same, accs[c], zeros16)
                            acc = acc + ai * rb_v[par, g * 16 + lane,
                                                  pl.ds(c * 16, 16)]
                            out_v[pl.ds(off + c * 16, 16)] = acc
                            accs[c] = acc
                        prev = di
                    return accs[0], accs[1], accs[2], accs[3], prev
                carry_in = lax.fori_loop(0, SB // 16, edgegrp, carry_in)
                cps = nxt
            return carry_in

        carry0 = (zeros16, zeros16, zeros16, zeros16, jnp.int32(-1))
        lax.fori_loop(0, nwin, passb_win, carry0)

        pltpu.sync_copy(
            out_v.at[pl.ds(0, NPS * D_H)],
            msg_h.at[pl.ds(pl.multiple_of(node_lo * D_H, 8), NPS * D_H)])

    return sck(xw_flat, qn_flat, kn_flat, e3, ebnd)


def _tc_aggregate(nr, bix, t):
    DT = NLAYERS * D_H
    t2 = jnp.reshape(t, (1, 1))

    def body1(nr_ref, b_ref, t_ref, am_ref):
        @pl.when(pl.program_id(0) == 0)
        def _():
            am_ref[...] = jnp.full((G, DT), NEG, _f32)
        xb = nr_ref[...] * t_ref[0, 0]
        b = b_ref[...]
        for g in range(G):
            m = jnp.max(jnp.where(b == g, xb, jnp.float32(NEG)),
                        axis=0, keepdims=True)
            am_ref[pl.ds(g, 1), :] = jnp.maximum(am_ref[pl.ds(g, 1), :], m)

    amax = pl.pallas_call(
        body1, grid=(NT // BN,),
        in_specs=[pl.BlockSpec((BN, DT), lambda i: (i, 0)),
                  pl.BlockSpec((BN, 1), lambda i: (i, 0)),
                  pl.BlockSpec((1, 1), lambda i: (0, 0))],
        out_specs=pl.BlockSpec((G, DT), lambda i: (0, 0)),
        out_shape=jax.ShapeDtypeStruct((G, DT), _f32),
    )(nr, bix, t2)

    def body2(nr_ref, b_ref, t_ref, am_ref, o_ref, sex_ref, sxex_ref):
        i = pl.program_id(0)

        @pl.when(i == 0)
        def _():
            sex_ref[...] = jnp.zeros((G, DT), _f32)
            sxex_ref[...] = jnp.zeros((G, DT), _f32)
        x = nr_ref[...]
        xb = x * t_ref[0, 0]
        b = b_ref[...]
        oh = (b == lax.broadcasted_iota(_i32, (BN, G), 1)).astype(_f32)
        am_rows = jnp.dot(oh, am_ref[...], preferred_element_type=_f32,
                          precision=lax.Precision.HIGHEST)
        ex = jnp.exp(xb - am_rows)
        dn = (((0,), (0,)), ((), ()))
        sex_ref[...] += lax.dot_general(
            oh, ex, dn, preferred_element_type=_f32,
            precision=lax.Precision.HIGHEST)
        sxex_ref[...] += lax.dot_general(
            oh, x * ex, dn, preferred_element_type=_f32,
            precision=lax.Precision.HIGHEST)

        @pl.when(i == NT // BN - 1)
        def _():
            o_ref[...] = sxex_ref[...] / (sex_ref[...] + jnp.float32(1e-16))

    return pl.pallas_call(
        body2, grid=(NT // BN,),
        in_specs=[pl.BlockSpec((BN, DT), lambda i: (i, 0)),
                  pl.BlockSpec((BN, 1), lambda i: (i, 0)),
                  pl.BlockSpec((1, 1), lambda i: (0, 0)),
                  pl.BlockSpec((G, DT), lambda i: (0, 0))],
        out_specs=pl.BlockSpec((G, DT), lambda i: (0, 0)),
        out_shape=jax.ShapeDtypeStruct((G, DT), _f32),
        scratch_shapes=[pltpu.VMEM((G, DT), _f32),
                        pltpu.VMEM((G, DT), _f32)],
    )(nr, bix, t2, amax)


def kernel(node_features, edge_index, edge_type, batch_index,
           W0, Ws, q_att, k_att, biases, t):
    src = edge_index[0]
    dst = edge_index[1]
    perm = jnp.argsort(dst)
    dst_s = dst[perm]
    src_s = src[perm]
    et_s = edge_type[perm]
    qidx = et_s * NT + dst_s
    kidx = et_s * NT + src_s
    padi = jnp.zeros((EPAD - E,), _i32)
    qidx = jnp.concatenate([qidx, padi])
    kidx = jnp.concatenate([kidx, padi])
    dst_p = jnp.concatenate([dst_s, jnp.full((EPAD - E,), 1 << 28, _i32)])
    e3 = jnp.stack([qidx, kidx, dst_p])
    ebnd = jnp.searchsorted(
        dst_s, jnp.minimum(jnp.arange(33, dtype=_i32) * NPS, N)).astype(_i32)
    ebnd = jnp.concatenate([ebnd, jnp.full((31,), E, _i32)])

    x0 = jnp.concatenate(
        [node_features, jnp.zeros((NT - N, node_features.shape[1]), _f32)])
    bix = jnp.concatenate(
        [batch_index, jnp.full((NT - N,), G, _i32)]).reshape(NT, 1)

    results = []
    xw, qn, kn, _ = _tc_layer(x0, W0, q_att[0], k_att[0], None)
    msg = _sc_layer(xw.reshape(R * NT, D_H), qn.reshape(-1), kn.reshape(-1),
                    e3, ebnd).reshape(NT, D_H)
    for i in range(NLAYERS - 1):
        xw, qn, kn, h = _tc_layer(msg, Ws[i], q_att[i + 1], k_att[i + 1],
                                  biases[i])
        results.append(h)
        msg = _sc_layer(xw.reshape(R * NT, D_H), qn.reshape(-1),
                        kn.reshape(-1), e3, ebnd).reshape(NT, D_H)
    results.append(_tc_bias(msg, biases[NLAYERS - 1]))
    nr_pad = jnp.concatenate(results, axis=-1)
    graph_representations = _tc_aggregate(nr_pad, bix, t)
    node_representations = nr_pad[:N]
    return (graph_representations, node_representations)


# speedup vs baseline: 35.7001x; 1.0000x over previous
import functools

import jax
import jax.numpy as jnp
from jax import lax
from jax.experimental import pallas as pl
from jax.experimental.pallas import tpu as pltpu
from jax.experimental.pallas import tpu_sc as plsc

N = 10000
E = 320000
D_H = 64
R = 3
G = 16
NLAYERS = 13
NW = 32
NPS = 320
NT = NW * NPS
BN = 512
WA = 2048
SB = 256
NSB = WA // SB
DUMP = NPS
OUTR = 336
EPAD = E + 2 * WA
NEG = -1e30

_f32 = jnp.float32
_i32 = jnp.int32


def _tc_layer(h, W, q, k, bias):
    in_d = W.shape[1]
    grid = (NT // BN,)
    q2 = q.reshape(1, D_H)
    k2 = k.reshape(1, D_H)
    have_bias = bias is not None

    def body(*refs):
        if have_bias:
            h_ref, w_ref, q_ref, k_ref, b_ref, xw_ref, qn_ref, kn_ref, ho_ref = refs
        else:
            h_ref, w_ref, q_ref, k_ref, xw_ref, qn_ref, kn_ref = refs
        hb = h_ref[...]
        if have_bias:
            hb = hb + b_ref[0]
            ho_ref[...] = hb
        qv = q_ref[0]
        kv = k_ref[0]
        for r in range(R):
            xwr = jnp.dot(hb, w_ref[r], preferred_element_type=_f32,
                          precision=lax.Precision.HIGHEST)
            xw_ref[r] = xwr
            qn_ref[r] = jnp.dot(xwr, qv, preferred_element_type=_f32,
                                precision=lax.Precision.HIGHEST)
            kn_ref[r] = jnp.dot(xwr, kv, preferred_element_type=_f32,
                                precision=lax.Precision.HIGHEST)

    in_specs = [
        pl.BlockSpec((BN, in_d), lambda i: (i, 0)),
        pl.BlockSpec((R, in_d, D_H), lambda i: (0, 0, 0)),
        pl.BlockSpec((1, D_H), lambda i: (0, 0)),
        pl.BlockSpec((1, D_H), lambda i: (0, 0)),
    ]
    args = [h, W, q2, k2]
    if have_bias:
        in_specs.append(pl.BlockSpec((1, D_H), lambda i: (0, 0)))
        args.append(bias.reshape(1, D_H))
    out_shape = [
        jax.ShapeDtypeStruct((R, NT, D_H), _f32),
        jax.ShapeDtypeStruct((R, NT), _f32),
        jax.ShapeDtypeStruct((R, NT), _f32),
    ]
    out_specs = [
        pl.BlockSpec((R, BN, D_H), lambda i: (0, i, 0)),
        pl.BlockSpec((R, BN), lambda i: (0, i)),
        pl.BlockSpec((R, BN), lambda i: (0, i)),
    ]
    if have_bias:
        out_shape.append(jax.ShapeDtypeStruct((NT, D_H), _f32))
        out_specs.append(pl.BlockSpec((BN, D_H), lambda i: (i, 0)))
    outs = pl.pallas_call(
        body, grid=grid, in_specs=in_specs, out_specs=out_specs,
        out_shape=out_shape)(*args)
    if have_bias:
        return outs[0], outs[1], outs[2], outs[3]
    return outs[0], outs[1], outs[2], None


def _tc_bias(msg, bias):
    def body(m_ref, b_ref, o_ref):
        o_ref[...] = m_ref[...] + b_ref[0]

    return pl.pallas_call(
        body, grid=(NT // BN,),
        in_specs=[pl.BlockSpec((BN, D_H), lambda i: (i, 0)),
                  pl.BlockSpec((1, D_H), lambda i: (0, 0))],
        out_specs=pl.BlockSpec((BN, D_H), lambda i: (i, 0)),
        out_shape=jax.ShapeDtypeStruct((NT, D_H), _f32),
    )(msg, bias.reshape(1, D_H))


def _seg_scan(v, key, kbuf, vbuf, op):
    kbuf[pl.ds(16, 16)] = key
    for s in (1, 2, 4, 8):
        vbuf[pl.ds(16, 16)] = v
        sv = vbuf[pl.ds(16 - s, 16)]
        sk = kbuf[pl.ds(16 - s, 16)]
        v = jnp.where(sk == key, op(v, sv), v)
    nxt = kbuf[pl.ds(17, 16)]
    return v, key != nxt


def _sc_layer(xw_flat, qn_flat, kn_flat, e3, ebnd):
    mesh = plsc.VectorSubcoreMesh(core_axis_name="c", subcore_axis_name="s")

    @functools.partial(
        pl.kernel, mesh=mesh,
        compiler_params=pltpu.CompilerParams(needs_layout_passes=False,
                                             use_tc_tiling_on_sc=False),
        out_type=jax.ShapeDtypeStruct((NT * D_H,), _f32),
        scratch_types=[
            pltpu.VMEM((R * NT,), _f32),
            pltpu.VMEM((R * NT,), _f32),
            pltpu.VMEM((64,), _i32),
            pltpu.VMEM((NPS,), _f32),
            pltpu.VMEM((NPS,), _f32),
            pltpu.VMEM((OUTR * D_H,), _f32),
            pltpu.VMEM((R, WA), _i32),
            pltpu.VMEM((48,), _f32),
            pltpu.VMEM((48,), _i32),
            pltpu.VMEM((SB,), _f32),
            pltpu.VMEM((2, SB, D_H), _f32),
            pltpu.SemaphoreType.DMA,
            pltpu.SemaphoreType.DMA,
        ],
    )
    def sck(xw_h, qn_h, kn_h, e3_h, ebnd_h, msg_h,
            qn_v, kn_v, ebnd_v, amax_v, den_v, out_v,
            s3_v, vbuf, kbuf, a_v, rb_v, sem0, sem1):
        wid = lax.axis_index("c") * 16 + lax.axis_index("s")
        node_lo = wid * NPS

        pltpu.sync_copy(qn_h, qn_v)
        pltpu.sync_copy(kn_h, kn_v)
        pltpu.sync_copy(ebnd_h, ebnd_v)
        eb = ebnd_v[pl.ds(wid, 16)]
        e_lo = eb[0]
        e_hi = eb[1]
        e0 = e_lo - lax.rem(e_lo, 8)

        zeros16 = jnp.zeros((16,), _f32)
        neg16 = jnp.full((16,), NEG, _f32)

        def init_small(i, _):
            amax_v[pl.ds(i * 16, 16)] = neg16
            den_v[pl.ds(i * 16, 16)] = zeros16
            return 0
        lax.fori_loop(0, NPS // 16, init_small, 0)

        def init_out(i, _):
            out_v[pl.ds(i * 16, 16)] = zeros16
            return 0
        lax.fori_loop(0, OUTR * D_H // 16, init_out, 0)

        kbuf[pl.ds(0, 16)] = jnp.full((16,), -1, _i32)
        kbuf[pl.ds(32, 16)] = jnp.full((16,), -2, _i32)

        iota16 = lax.iota(_i32, 16)
        nwin = (e_hi - e0 + (WA - 1)) // WA

        def stage(estart):
            estart = pl.multiple_of(estart, 8)
            pltpu.sync_copy(e3_h.at[:, pl.ds(estart, WA)], s3_v)

        def alpha_of(base, estart):
            qi = plsc.load_gather(qn_v, [s3_v[0, pl.ds(base, 16)]])
            kj = plsc.load_gather(kn_v, [s3_v[1, pl.ds(base, 16)]])
            s = qi + kj
            alpha = jnp.where(s >= 0.0, s, s * jnp.float32(0.2))
            ev = estart + base + iota16
            valid = (ev >= e_lo) & (ev < e_hi)
            key = s3_v[2, pl.ds(base, 16)]
            kloc = jnp.clip(key - node_lo, 0, NPS - 1)
            return alpha, valid, key, kloc

        vbuf[pl.ds(0, 16)] = neg16

        def pass1_win(j, _):
            estart = e0 + j * WA
            stage(estart)

            def vec(v, _):
                base = v * 16
                alpha, valid, key, kloc = alpha_of(base, estart)
                alpha = jnp.where(valid, alpha, jnp.float32(NEG))
                sm, last = _seg_scan(alpha, key, kbuf, vbuf, jnp.maximum)
                cur = plsc.load_gather(amax_v, [kloc])
                plsc.store_scatter(amax_v, [kloc], jnp.maximum(cur, sm),
                                   mask=last & valid)
                return 0
            lax.fori_loop(0, WA // 16, vec, 0)
            return 0
        lax.fori_loop(0, nwin, pass1_win, 0)

        vbuf[pl.ds(0, 16)] = zeros16

        def pass2_win(j, _):
            estart = e0 + j * WA
            stage(estart)

            def vec(v, _):
                base = v * 16
                alpha, valid, key, kloc = alpha_of(base, estart)
                am = plsc.load_gather(amax_v, [kloc])
                ex = jnp.exp(jnp.where(valid, alpha - am, jnp.float32(NEG)))
                ss, last = _seg_scan(ex, key, kbuf, vbuf, lambda a, b: a + b)
                cur = plsc.load_gather(den_v, [kloc])
                plsc.store_scatter(den_v, [kloc], cur + ss,
                                   mask=last & valid)
                return 0
            lax.fori_loop(0, WA // 16, vec, 0)
            return 0
        lax.fori_loop(0, nwin, pass2_win, 0)

        sems = (sem0, sem1)

        def fire(s):
            par = s % 2
            sbase = s * SB
            cps = []
            for h in range(SB // 128):
                idx_ref = s3_v.at[1, pl.ds(sbase + h * 128, 128)]
                dst_ref = rb_v.at[par, pl.ds(h * 128, 128), :]
                cps.append(pltpu.async_copy(xw_h.at[idx_ref], dst_ref,
                                            sems[par]))
            return cps

        def passb_win(j, carry):
            estart = e0 + j * WA
            stage(estart)
            carry_in = carry
            cps = fire(0)
            for s in range(NSB):
                par = s % 2
                sbase = s * SB
                nxt = fire(s + 1) if s + 1 < NSB else None
                def avec(v, _):
                    base = sbase + v * 16
                    alpha, valid, key, kloc = alpha_of(base, estart)
                    am = plsc.load_gather(amax_v, [kloc])
                    den = plsc.load_gather(den_v, [kloc])
                    ex = jnp.exp(jnp.where(valid, alpha - am,
                                           jnp.float32(NEG)))
                    a = ex / (den + jnp.float32(1e-16))
                    a_v[pl.ds(v * 16, 16)] = jnp.where(valid, a,
                                                       jnp.float32(0.0))
                    return 0
                lax.fori_loop(0, SB // 16, avec, 0)
                for cp in cps:
                    cp.wait()

                def edgegrp(g, car):
                    acc0, acc1, acc2, acc3, prev = car
                    accs = [acc0, acc1, acc2, acc3]
                    a16 = a_v[pl.ds(g * 16, 16)]
                    d16 = s3_v[2, pl.ds(sbase + g * 16, 16)]
                    kbuf[pl.ds(16, 16)] = d16
                    sh = kbuf[pl.ds(15, 16)]
                    samef = jnp.where(d16 == sh, jnp.float32(1.0),
                                      jnp.float32(0.0))
                    dl16 = d16 - node_lo
                    ok16 = (dl16 >= 0) & (dl16 < NPS)
                    off16 = jnp.where(ok16, dl16, DUMP) * D_H
                    same0 = jnp.where(d16[0] == prev, jnp.float32(1.0),
                                      jnp.float32(0.0))
                    for lane in range(16):
                        ai = a16[lane]
                        sf = same0 if lane == 0 else samef[lane]
                        off = off16[lane]
                        for c in range(D_H // 16):
                            acc = sf * accs[c] + ai * rb_v[
                                par, g * 16 + lane, pl.ds(c * 16, 16)]
                            out_v[pl.ds(off + c * 16, 16)] = acc
                            accs[c] = acc
                    return accs[0], accs[1], accs[2], accs[3], d16[15]
                carry_in = lax.fori_loop(0, SB // 16, edgegrp, carry_in)
                cps = nxt
            return carry_in

        carry0 = (zeros16, zeros16, zeros16, zeros16, jnp.int32(-1))
        lax.fori_loop(0, nwin, passb_win, carry0)

        pltpu.sync_copy(
            out_v.at[pl.ds(0, NPS * D_H)],
            msg_h.at[pl.ds(pl.multiple_of(node_lo * D_H, 8), NPS * D_H)])

    return sck(xw_flat, qn_flat, kn_flat, e3, ebnd)


def _tc_aggregate(nr, bix, t):
    DT = NLAYERS * D_H
    t2 = jnp.reshape(t, (1, 1))

    def body1(nr_ref, b_ref, t_ref, am_ref):
        @pl.when(pl.program_id(0) == 0)
        def _():
            am_ref[...] = jnp.full((G, DT), NEG, _f32)
        xb = nr_ref[...] * t_ref[0, 0]
        b = b_ref[...]
        for g in range(G):
            m = jnp.max(jnp.where(b == g, xb, jnp.float32(NEG)),
                        axis=0, keepdims=True)
            am_ref[pl.ds(g, 1), :] = jnp.maximum(am_ref[pl.ds(g, 1), :], m)

    amax = pl.pallas_call(
        body1, grid=(NT // BN,),
        in_specs=[pl.BlockSpec((BN, DT), lambda i: (i, 0)),
                  pl.BlockSpec((BN, 1), lambda i: (i, 0)),
                  pl.BlockSpec((1, 1), lambda i: (0, 0))],
        out_specs=pl.BlockSpec((G, DT), lambda i: (0, 0)),
        out_shape=jax.ShapeDtypeStruct((G, DT), _f32),
    )(nr, bix, t2)

    def body2(nr_ref, b_ref, t_ref, am_ref, o_ref, sex_ref, sxex_ref):
        i = pl.program_id(0)

        @pl.when(i == 0)
        def _():
            sex_ref[...] = jnp.zeros((G, DT), _f32)
            sxex_ref[...] = jnp.zeros((G, DT), _f32)
        x = nr_ref[...]
        xb = x * t_ref[0, 0]
        b = b_ref[...]
        oh = (b == lax.broadcasted_iota(_i32, (BN, G), 1)).astype(_f32)
        am_rows = jnp.dot(oh, am_ref[...], preferred_element_type=_f32,
                          precision=lax.Precision.HIGHEST)
        ex = jnp.exp(xb - am_rows)
        dn = (((0,), (0,)), ((), ()))
        sex_ref[...] += lax.dot_general(
            oh, ex, dn, preferred_element_type=_f32,
            precision=lax.Precision.HIGHEST)
        sxex_ref[...] += lax.dot_general(
            oh, x * ex, dn, preferred_element_type=_f32,
            precision=lax.Precision.HIGHEST)

        @pl.when(i == NT // BN - 1)
        def _():
            o_ref[...] = sxex_ref[...] / (sex_ref[...] + jnp.float32(1e-16))

    return pl.pallas_call(
        body2, grid=(NT // BN,),
        in_specs=[pl.BlockSpec((BN, DT), lambda i: (i, 0)),
                  pl.BlockSpec((BN, 1), lambda i: (i, 0)),
                  pl.BlockSpec((1, 1), lambda i: (0, 0)),
                  pl.BlockSpec((G, DT), lambda i: (0, 0))],
        out_specs=pl.BlockSpec((G, DT), lambda i: (0, 0)),
        out_shape=jax.ShapeDtypeStruct((G, DT), _f32),
        scratch_shapes=[pltpu.VMEM((G, DT), _f32),
                        pltpu.VMEM((G, DT), _f32)],
    )(nr, bix, t2, amax)


def kernel(node_features, edge_index, edge_type, batch_index,
           W0, Ws, q_att, k_att, biases, t):
    src = edge_index[0]
    dst = edge_index[1]
    perm = jnp.argsort(dst)
    dst_s = dst[perm]
    src_s = src[perm]
    et_s = edge_type[perm]
    qidx = et_s * NT + dst_s
    kidx = et_s * NT + src_s
    padi = jnp.zeros((EPAD - E,), _i32)
    qidx = jnp.concatenate([qidx, padi])
    kidx = jnp.concatenate([kidx, padi])
    dst_p = jnp.concatenate([dst_s, jnp.full((EPAD - E,), 1 << 28, _i32)])
    e3 = jnp.stack([qidx, kidx, dst_p])
    ebnd = jnp.searchsorted(
        dst_s, jnp.minimum(jnp.arange(33, dtype=_i32) * NPS, N)).astype(_i32)
    ebnd = jnp.concatenate([ebnd, jnp.full((31,), E, _i32)])

    x0 = jnp.concatenate(
        [node_features, jnp.zeros((NT - N, node_features.shape[1]), _f32)])
    bix = jnp.concatenate(
        [batch_index, jnp.full((NT - N,), G, _i32)]).reshape(NT, 1)

    results = []
    xw, qn, kn, _ = _tc_layer(x0, W0, q_att[0], k_att[0], None)
    msg = _sc_layer(xw.reshape(R * NT, D_H), qn.reshape(-1), kn.reshape(-1),
                    e3, ebnd).reshape(NT, D_H)
    for i in range(NLAYERS - 1):
        xw, qn, kn, h = _tc_layer(msg, Ws[i], q_att[i + 1], k_att[i + 1],
                                  biases[i])
        results.append(h)
        msg = _sc_layer(xw.reshape(R * NT, D_H), qn.reshape(-1),
                        kn.reshape(-1), e3, ebnd).reshape(NT, D_H)
    results.append(_tc_bias(msg, biases[NLAYERS - 1]))
    nr_pad = jnp.concatenate(results, axis=-1)
    graph_representations = _tc_aggregate(nr_pad, bix, t)
    node_representations = nr_pad[:N]
    return (graph_representations, node_representations)


# speedup vs baseline: 36.2827x; 1.0163x over previous
import functools

import jax
import jax.numpy as jnp
from jax import lax
from jax.experimental import pallas as pl
from jax.experimental.pallas import tpu as pltpu
from jax.experimental.pallas import tpu_sc as plsc

N = 10000
E = 320000
D_H = 64
R = 3
G = 16
NLAYERS = 13
NW = 32
NPS = 320
NT = NW * NPS
BN = 512
WA = 2048
SB = 256
NSB = WA // SB
DUMP = NPS
OUTR = 336
EPAD = E + 2 * WA
NEG = -1e30

_f32 = jnp.float32
_i32 = jnp.int32


def _tc_layer(h, W, q, k, bias):
    in_d = W.shape[1]
    grid = (NT // BN,)
    q2 = q.reshape(1, D_H)
    k2 = k.reshape(1, D_H)
    have_bias = bias is not None

    def body(*refs):
        if have_bias:
            h_ref, w_ref, q_ref, k_ref, b_ref, xw_ref, qn_ref, kn_ref, ho_ref = refs
        else:
            h_ref, w_ref, q_ref, k_ref, xw_ref, qn_ref, kn_ref = refs
        hb = h_ref[...]
        if have_bias:
            hb = hb + b_ref[0]
            ho_ref[...] = hb
        qv = q_ref[0]
        kv = k_ref[0]
        for r in range(R):
            xwr = jnp.dot(hb, w_ref[r], preferred_element_type=_f32,
                          precision=lax.Precision.HIGHEST)
            xw_ref[r] = xwr
            qn_ref[r] = jnp.dot(xwr, qv, preferred_element_type=_f32,
                                precision=lax.Precision.HIGHEST)
            kn_ref[r] = jnp.dot(xwr, kv, preferred_element_type=_f32,
                                precision=lax.Precision.HIGHEST)

    in_specs = [
        pl.BlockSpec((BN, in_d), lambda i: (i, 0)),
        pl.BlockSpec((R, in_d, D_H), lambda i: (0, 0, 0)),
        pl.BlockSpec((1, D_H), lambda i: (0, 0)),
        pl.BlockSpec((1, D_H), lambda i: (0, 0)),
    ]
    args = [h, W, q2, k2]
    if have_bias:
        in_specs.append(pl.BlockSpec((1, D_H), lambda i: (0, 0)))
        args.append(bias.reshape(1, D_H))
    out_shape = [
        jax.ShapeDtypeStruct((R, NT, D_H), _f32),
        jax.ShapeDtypeStruct((R, NT), _f32),
        jax.ShapeDtypeStruct((R, NT), _f32),
    ]
    out_specs = [
        pl.BlockSpec((R, BN, D_H), lambda i: (0, i, 0)),
        pl.BlockSpec((R, BN), lambda i: (0, i)),
        pl.BlockSpec((R, BN), lambda i: (0, i)),
    ]
    if have_bias:
        out_shape.append(jax.ShapeDtypeStruct((NT, D_H), _f32))
        out_specs.append(pl.BlockSpec((BN, D_H), lambda i: (i, 0)))
    outs = pl.pallas_call(
        body, grid=grid, in_specs=in_specs, out_specs=out_specs,
        out_shape=out_shape)(*args)
    if have_bias:
        return outs[0], outs[1], outs[2], outs[3]
    return outs[0], outs[1], outs[2], None


def _tc_bias(msg, bias):
    def body(m_ref, b_ref, o_ref):
        o_ref[...] = m_ref[...] + b_ref[0]

    return pl.pallas_call(
        body, grid=(NT // BN,),
        in_specs=[pl.BlockSpec((BN, D_H), lambda i: (i, 0)),
                  pl.BlockSpec((1, D_H), lambda i: (0, 0))],
        out_specs=pl.BlockSpec((BN, D_H), lambda i: (i, 0)),
        out_shape=jax.ShapeDtypeStruct((NT, D_H), _f32),
    )(msg, bias.reshape(1, D_H))


def _seg_scan(v, key, kbuf, vbuf, op):
    kbuf[pl.ds(16, 16)] = key
    for s in (1, 2, 4, 8):
        vbuf[pl.ds(16, 16)] = v
        sv = vbuf[pl.ds(16 - s, 16)]
        sk = kbuf[pl.ds(16 - s, 16)]
        v = jnp.where(sk == key, op(v, sv), v)
    nxt = kbuf[pl.ds(17, 16)]
    return v, key != nxt


def _sc_layer(xw_flat, qn_flat, kn_flat, e3, ebnd):
    mesh = plsc.VectorSubcoreMesh(core_axis_name="c", subcore_axis_name="s")

    @functools.partial(
        pl.kernel, mesh=mesh,
        compiler_params=pltpu.CompilerParams(needs_layout_passes=False,
                                             use_tc_tiling_on_sc=False),
        out_type=jax.ShapeDtypeStruct((NT * D_H,), _f32),
        scratch_types=[
            pltpu.VMEM((R * NT,), _f32),
            pltpu.VMEM((R * NT,), _f32),
            pltpu.VMEM((64,), _i32),
            pltpu.VMEM((NPS,), _f32),
            pltpu.VMEM((NPS,), _f32),
            pltpu.VMEM((OUTR * D_H,), _f32),
            pltpu.VMEM((R, WA), _i32),
            pltpu.VMEM((48,), _f32),
            pltpu.VMEM((48,), _i32),
            pltpu.VMEM((SB,), _f32),
            pltpu.VMEM((2, SB, D_H), _f32),
            pltpu.SemaphoreType.DMA,
            pltpu.SemaphoreType.DMA,
        ],
    )
    def sck(xw_h, qn_h, kn_h, e3_h, ebnd_h, msg_h,
            qn_v, kn_v, ebnd_v, amax_v, den_v, out_v,
            s3_v, vbuf, kbuf, a_v, rb_v, sem0, sem1):
        wid = lax.axis_index("c") * 16 + lax.axis_index("s")
        node_lo = wid * NPS

        pltpu.sync_copy(qn_h, qn_v)
        pltpu.sync_copy(kn_h, kn_v)
        pltpu.sync_copy(ebnd_h, ebnd_v)
        eb = ebnd_v[pl.ds(wid, 16)]
        e_lo = eb[0]
        e_hi = eb[1]
        e0 = e_lo - lax.rem(e_lo, 8)

        zeros16 = jnp.zeros((16,), _f32)
        neg16 = jnp.full((16,), NEG, _f32)

        def init_small(i, _):
            amax_v[pl.ds(i * 16, 16)] = neg16
            den_v[pl.ds(i * 16, 16)] = zeros16
            return 0
        lax.fori_loop(0, NPS // 16, init_small, 0)

        def init_out(i, _):
            out_v[pl.ds(i * 16, 16)] = zeros16
            return 0
        lax.fori_loop(0, OUTR * D_H // 16, init_out, 0)

        kbuf[pl.ds(0, 16)] = jnp.full((16,), -1, _i32)
        kbuf[pl.ds(32, 16)] = jnp.full((16,), -2, _i32)

        iota16 = lax.iota(_i32, 16)
        nwin = (e_hi - e0 + (WA - 1)) // WA

        def stage(estart):
            estart = pl.multiple_of(estart, 8)
            pltpu.sync_copy(e3_h.at[:, pl.ds(estart, WA)], s3_v)

        def alpha_of(base, estart):
            qi = plsc.load_gather(qn_v, [s3_v[0, pl.ds(base, 16)]])
            kj = plsc.load_gather(kn_v, [s3_v[1, pl.ds(base, 16)]])
            s = qi + kj
            alpha = jnp.where(s >= 0.0, s, s * jnp.float32(0.2))
            ev = estart + base + iota16
            valid = (ev >= e_lo) & (ev < e_hi)
            key = s3_v[2, pl.ds(base, 16)]
            kloc = jnp.clip(key - node_lo, 0, NPS - 1)
            return alpha, valid, key, kloc

        vbuf[pl.ds(0, 16)] = neg16

        def pass1_win(j, _):
            estart = e0 + j * WA
            stage(estart)

            def vec(v, _):
                base = v * 16
                alpha, valid, key, kloc = alpha_of(base, estart)
                alpha = jnp.where(valid, alpha, jnp.float32(NEG))
                sm, last = _seg_scan(alpha, key, kbuf, vbuf, jnp.maximum)
                cur = plsc.load_gather(amax_v, [kloc])
                plsc.store_scatter(amax_v, [kloc], jnp.maximum(cur, sm),
                                   mask=last & valid)
                return 0
            lax.fori_loop(0, WA // 16, vec, 0)
            return 0
        lax.fori_loop(0, nwin, pass1_win, 0)

        vbuf[pl.ds(0, 16)] = zeros16

        def pass2_win(j, _):
            estart = e0 + j * WA
            stage(estart)

            def vec(v, _):
                base = v * 16
                alpha, valid, key, kloc = alpha_of(base, estart)
                am = plsc.load_gather(amax_v, [kloc])
                ex = jnp.exp(jnp.where(valid, alpha - am, jnp.float32(NEG)))
                plsc.addupdate_scatter(den_v, [kloc], ex, mask=valid)
                return 0
            lax.fori_loop(0, WA // 16, vec, 0)
            return 0
        lax.fori_loop(0, nwin, pass2_win, 0)

        sems = (sem0, sem1)

        def fire(s):
            par = s % 2
            sbase = s * SB
            cps = []
            for h in range(SB // 128):
                idx_ref = s3_v.at[1, pl.ds(sbase + h * 128, 128)]
                dst_ref = rb_v.at[par, pl.ds(h * 128, 128), :]
                cps.append(pltpu.async_copy(xw_h.at[idx_ref], dst_ref,
                                            sems[par]))
            return cps

        def passb_win(j, carry):
            estart = e0 + j * WA
            stage(estart)
            carry_in = carry
            cps = fire(0)
            for s in range(NSB):
                par = s % 2
                sbase = s * SB
                nxt = fire(s + 1) if s + 1 < NSB else None
                def avec(v, _):
                    base = sbase + v * 16
                    alpha, valid, key, kloc = alpha_of(base, estart)
                    am = plsc.load_gather(amax_v, [kloc])
                    den = plsc.load_gather(den_v, [kloc])
                    ex = jnp.exp(jnp.where(valid, alpha - am,
                                           jnp.float32(NEG)))
                    a = ex / (den + jnp.float32(1e-16))
                    a_v[pl.ds(v * 16, 16)] = jnp.where(valid, a,
                                                       jnp.float32(0.0))
                    return 0
                lax.fori_loop(0, SB // 16, avec, 0)
                for cp in cps:
                    cp.wait()

                def edgegrp(g, car):
                    acc0, acc1, acc2, acc3, prev = car
                    accs = [acc0, acc1, acc2, acc3]
                    a16 = a_v[pl.ds(g * 16, 16)]
                    d16 = s3_v[2, pl.ds(sbase + g * 16, 16)]
                    kbuf[pl.ds(16, 16)] = d16
                    sh = kbuf[pl.ds(15, 16)]
                    samef = jnp.where(d16 == sh, jnp.float32(1.0),
                                      jnp.float32(0.0))
                    dl16 = d16 - node_lo
                    ok16 = (dl16 >= 0) & (dl16 < NPS)
                    off16 = jnp.where(ok16, dl16, DUMP) * D_H
                    same0 = jnp.where(d16[0] == prev, jnp.float32(1.0),
                                      jnp.float32(0.0))
                    for lane in range(16):
                        ai = a16[lane]
                        sf = same0 if lane == 0 else samef[lane]
                        off = off16[lane]
                        for c in range(D_H // 16):
                            acc = sf * accs[c] + ai * rb_v[
                                par, g * 16 + lane, pl.ds(c * 16, 16)]
                            out_v[pl.ds(off + c * 16, 16)] = acc
                            accs[c] = acc
                    return accs[0], accs[1], accs[2], accs[3], d16[15]
                carry_in = lax.fori_loop(0, SB // 16, edgegrp, carry_in)
                cps = nxt
            return carry_in

        carry0 = (zeros16, zeros16, zeros16, zeros16, jnp.int32(-1))
        lax.fori_loop(0, nwin, passb_win, carry0)

        pltpu.sync_copy(
            out_v.at[pl.ds(0, NPS * D_H)],
            msg_h.at[pl.ds(pl.multiple_of(node_lo * D_H, 8), NPS * D_H)])

    return sck(xw_flat, qn_flat, kn_flat, e3, ebnd)


def _tc_aggregate(nr, bix, t):
    DT = NLAYERS * D_H
    t2 = jnp.reshape(t, (1, 1))

    def body1(nr_ref, b_ref, t_ref, am_ref):
        @pl.when(pl.program_id(0) == 0)
        def _():
            am_ref[...] = jnp.full((G, DT), NEG, _f32)
        xb = nr_ref[...] * t_ref[0, 0]
        b = b_ref[...]
        for g in range(G):
            m = jnp.max(jnp.where(b == g, xb, jnp.float32(NEG)),
                        axis=0, keepdims=True)
            am_ref[pl.ds(g, 1), :] = jnp.maximum(am_ref[pl.ds(g, 1), :], m)

    amax = pl.pallas_call(
        body1, grid=(NT // BN,),
        in_specs=[pl.BlockSpec((BN, DT), lambda i: (i, 0)),
                  pl.BlockSpec((BN, 1), lambda i: (i, 0)),
                  pl.BlockSpec((1, 1), lambda i: (0, 0))],
        out_specs=pl.BlockSpec((G, DT), lambda i: (0, 0)),
        out_shape=jax.ShapeDtypeStruct((G, DT), _f32),
    )(nr, bix, t2)

    def body2(nr_ref, b_ref, t_ref, am_ref, o_ref, sex_ref, sxex_ref):
        i = pl.program_id(0)

        @pl.when(i == 0)
        def _():
            sex_ref[...] = jnp.zeros((G, DT), _f32)
            sxex_ref[...] = jnp.zeros((G, DT), _f32)
        x = nr_ref[...]
        xb = x * t_ref[0, 0]
        b = b_ref[...]
        oh = (b == lax.broadcasted_iota(_i32, (BN, G), 1)).astype(_f32)
        am_rows = jnp.dot(oh, am_ref[...], preferred_element_type=_f32,
                          precision=lax.Precision.HIGHEST)
        ex = jnp.exp(xb - am_rows)
        dn = (((0,), (0,)), ((), ()))
        sex_ref[...] += lax.dot_general(
            oh, ex, dn, preferred_element_type=_f32,
            precision=lax.Precision.HIGHEST)
        sxex_ref[...] += lax.dot_general(
            oh, x * ex, dn, preferred_element_type=_f32,
            precision=lax.Precision.HIGHEST)

        @pl.when(i == NT // BN - 1)
        def _():
            o_ref[...] = sxex_ref[...] / (sex_ref[...] + jnp.float32(1e-16))

    return pl.pallas_call(
        body2, grid=(NT // BN,),
        in_specs=[pl.BlockSpec((BN, DT), lambda i: (i, 0)),
                  pl.BlockSpec((BN, 1), lambda i: (i, 0)),
                  pl.BlockSpec((1, 1), lambda i: (0, 0)),
                  pl.BlockSpec((G, DT), lambda i: (0, 0))],
        out_specs=pl.BlockSpec((G, DT), lambda i: (0, 0)),
        out_shape=jax.ShapeDtypeStruct((G, DT), _f32),
        scratch_shapes=[pltpu.VMEM((G, DT), _f32),
                        pltpu.VMEM((G, DT), _f32)],
    )(nr, bix, t2, amax)


def kernel(node_features, edge_index, edge_type, batch_index,
           W0, Ws, q_att, k_att, biases, t):
    src = edge_index[0]
    dst = edge_index[1]
    perm = jnp.argsort(dst)
    dst_s = dst[perm]
    src_s = src[perm]
    et_s = edge_type[perm]
    qidx = et_s * NT + dst_s
    kidx = et_s * NT + src_s
    padi = jnp.zeros((EPAD - E,), _i32)
    qidx = jnp.concatenate([qidx, padi])
    kidx = jnp.concatenate([kidx, padi])
    dst_p = jnp.concatenate([dst_s, jnp.full((EPAD - E,), 1 << 28, _i32)])
    e3 = jnp.stack([qidx, kidx, dst_p])
    ebnd = jnp.searchsorted(
        dst_s, jnp.minimum(jnp.arange(33, dtype=_i32) * NPS, N)).astype(_i32)
    ebnd = jnp.concatenate([ebnd, jnp.full((31,), E, _i32)])

    x0 = jnp.concatenate(
        [node_features, jnp.zeros((NT - N, node_features.shape[1]), _f32)])
    bix = jnp.concatenate(
        [batch_index, jnp.full((NT - N,), G, _i32)]).reshape(NT, 1)

    results = []
    xw, qn, kn, _ = _tc_layer(x0, W0, q_att[0], k_att[0], None)
    msg = _sc_layer(xw.reshape(R * NT, D_H), qn.reshape(-1), kn.reshape(-1),
                    e3, ebnd).reshape(NT, D_H)
    for i in range(NLAYERS - 1):
        xw, qn, kn, h = _tc_layer(msg, Ws[i], q_att[i + 1], k_att[i + 1],
                                  biases[i])
        results.append(h)
        msg = _sc_layer(xw.reshape(R * NT, D_H), qn.reshape(-1),
                        kn.reshape(-1), e3, ebnd).reshape(NT, D_H)
    results.append(_tc_bias(msg, biases[NLAYERS - 1]))
    nr_pad = jnp.concatenate(results, axis=-1)
    graph_representations = _tc_aggregate(nr_pad, bix, t)
    node_representations = nr_pad[:N]
    return (graph_representations, node_representations)


# speedup vs baseline: 36.3141x; 1.0009x over previous
import functools

import jax
import jax.numpy as jnp
from jax import lax
from jax.experimental import pallas as pl
from jax.experimental.pallas import tpu as pltpu
from jax.experimental.pallas import tpu_sc as plsc

N = 10000
E = 320000
D_H = 64
R = 3
G = 16
NLAYERS = 13
NW = 32
NPS = 320
NT = NW * NPS
BN = 512
WA = 2048
SB = 256
NSB = WA // SB
DUMP = NPS
OUTR = 336
EPAD = E + 2 * WA
NEG = -1e30

_f32 = jnp.float32
_i32 = jnp.int32


def _tc_layer(h, W, q, k, bias):
    in_d = W.shape[1]
    grid = (NT // BN,)
    q2 = q.reshape(1, D_H)
    k2 = k.reshape(1, D_H)
    have_bias = bias is not None

    def body(*refs):
        if have_bias:
            h_ref, w_ref, q_ref, k_ref, b_ref, xw_ref, qn_ref, kn_ref, ho_ref = refs
        else:
            h_ref, w_ref, q_ref, k_ref, xw_ref, qn_ref, kn_ref = refs
        hb = h_ref[...]
        if have_bias:
            hb = hb + b_ref[0]
            ho_ref[...] = hb
        qv = q_ref[0]
        kv = k_ref[0]
        for r in range(R):
            xwr = jnp.dot(hb, w_ref[r], preferred_element_type=_f32,
                          precision=lax.Precision.HIGHEST)
            xw_ref[r] = xwr
            qn_ref[r] = jnp.dot(xwr, qv, preferred_element_type=_f32,
                                precision=lax.Precision.HIGHEST)
            kn_ref[r] = jnp.dot(xwr, kv, preferred_element_type=_f32,
                                precision=lax.Precision.HIGHEST)

    in_specs = [
        pl.BlockSpec((BN, in_d), lambda i: (i, 0)),
        pl.BlockSpec((R, in_d, D_H), lambda i: (0, 0, 0)),
        pl.BlockSpec((1, D_H), lambda i: (0, 0)),
        pl.BlockSpec((1, D_H), lambda i: (0, 0)),
    ]
    args = [h, W, q2, k2]
    if have_bias:
        in_specs.append(pl.BlockSpec((1, D_H), lambda i: (0, 0)))
        args.append(bias.reshape(1, D_H))
    out_shape = [
        jax.ShapeDtypeStruct((R, NT, D_H), _f32),
        jax.ShapeDtypeStruct((R, NT), _f32),
        jax.ShapeDtypeStruct((R, NT), _f32),
    ]
    out_specs = [
        pl.BlockSpec((R, BN, D_H), lambda i: (0, i, 0)),
        pl.BlockSpec((R, BN), lambda i: (0, i)),
        pl.BlockSpec((R, BN), lambda i: (0, i)),
    ]
    if have_bias:
        out_shape.append(jax.ShapeDtypeStruct((NT, D_H), _f32))
        out_specs.append(pl.BlockSpec((BN, D_H), lambda i: (i, 0)))
    outs = pl.pallas_call(
        body, grid=grid, in_specs=in_specs, out_specs=out_specs,
        out_shape=out_shape)(*args)
    if have_bias:
        return outs[0], outs[1], outs[2], outs[3]
    return outs[0], outs[1], outs[2], None


def _tc_bias(msg, bias):
    def body(m_ref, b_ref, o_ref):
        o_ref[...] = m_ref[...] + b_ref[0]

    return pl.pallas_call(
        body, grid=(NT // BN,),
        in_specs=[pl.BlockSpec((BN, D_H), lambda i: (i, 0)),
                  pl.BlockSpec((1, D_H), lambda i: (0, 0))],
        out_specs=pl.BlockSpec((BN, D_H), lambda i: (i, 0)),
        out_shape=jax.ShapeDtypeStruct((NT, D_H), _f32),
    )(msg, bias.reshape(1, D_H))


def _seg_scan(v, key, kbuf, vbuf, op):
    kbuf[pl.ds(16, 16)] = key
    for s in (1, 2, 4, 8):
        vbuf[pl.ds(16, 16)] = v
        sv = vbuf[pl.ds(16 - s, 16)]
        sk = kbuf[pl.ds(16 - s, 16)]
        v = jnp.where(sk == key, op(v, sv), v)
    nxt = kbuf[pl.ds(17, 16)]
    return v, key != nxt


def _sc_layer(xw_flat, qn_flat, kn_flat, e3, ebnd):
    mesh = plsc.VectorSubcoreMesh(core_axis_name="c", subcore_axis_name="s")

    @functools.partial(
        pl.kernel, mesh=mesh,
        compiler_params=pltpu.CompilerParams(needs_layout_passes=False,
                                             use_tc_tiling_on_sc=False),
        out_type=jax.ShapeDtypeStruct((NT * D_H,), _f32),
        scratch_types=[
            pltpu.VMEM((R * NT,), _f32),
            pltpu.VMEM((R * NT,), _f32),
            pltpu.VMEM((64,), _i32),
            pltpu.VMEM((NPS,), _f32),
            pltpu.VMEM((NPS,), _f32),
            pltpu.VMEM((OUTR * D_H,), _f32),
            pltpu.VMEM((R, WA), _i32),
            pltpu.VMEM((48,), _f32),
            pltpu.VMEM((48,), _i32),
            pltpu.VMEM((SB,), _f32),
            pltpu.VMEM((2, SB, D_H), _f32),
            pltpu.SemaphoreType.DMA,
            pltpu.SemaphoreType.DMA,
        ],
    )
    def sck(xw_h, qn_h, kn_h, e3_h, ebnd_h, msg_h,
            qn_v, kn_v, ebnd_v, amax_v, den_v, out_v,
            s3_v, vbuf, kbuf, a_v, rb_v, sem0, sem1):
        wid = lax.axis_index("c") * 16 + lax.axis_index("s")
        node_lo = wid * NPS

        pltpu.sync_copy(qn_h, qn_v)
        pltpu.sync_copy(kn_h, kn_v)
        pltpu.sync_copy(ebnd_h, ebnd_v)
        eb = ebnd_v[pl.ds(wid, 16)]
        e_lo = eb[0]
        e_hi = eb[1]
        e0 = e_lo - lax.rem(e_lo, 8)

        zeros16 = jnp.zeros((16,), _f32)
        neg16 = jnp.full((16,), NEG, _f32)

        def init_small(i, _):
            amax_v[pl.ds(i * 16, 16)] = neg16
            den_v[pl.ds(i * 16, 16)] = zeros16
            return 0
        lax.fori_loop(0, NPS // 16, init_small, 0)

        def init_out(i, _):
            out_v[pl.ds(i * 16, 16)] = zeros16
            return 0
        lax.fori_loop(0, OUTR * D_H // 16, init_out, 0)

        kbuf[pl.ds(0, 16)] = jnp.full((16,), -1, _i32)
        kbuf[pl.ds(32, 16)] = jnp.full((16,), -2, _i32)

        iota16 = lax.iota(_i32, 16)
        nwin = (e_hi - e0 + (WA - 1)) // WA

        def stage(estart):
            estart = pl.multiple_of(estart, 8)
            pltpu.sync_copy(e3_h.at[:, pl.ds(estart, WA)], s3_v)

        def alpha_of(base, estart):
            qi = plsc.load_gather(qn_v, [s3_v[0, pl.ds(base, 16)]])
            kj = plsc.load_gather(kn_v, [s3_v[1, pl.ds(base, 16)]])
            s = qi + kj
            alpha = jnp.where(s >= 0.0, s, s * jnp.float32(0.2))
            ev = estart + base + iota16
            valid = (ev >= e_lo) & (ev < e_hi)
            key = s3_v[2, pl.ds(base, 16)]
            kloc = jnp.clip(key - node_lo, 0, NPS - 1)
            return alpha, valid, key, kloc

        vbuf[pl.ds(0, 16)] = neg16

        def pass1_win(j, _):
            estart = e0 + j * WA
            stage(estart)

            def vec(v, _):
                base = v * 16
                alpha, valid, key, kloc = alpha_of(base, estart)
                alpha = jnp.where(valid, alpha, jnp.float32(NEG))
                sm, last = _seg_scan(alpha, key, kbuf, vbuf, jnp.maximum)
                cur = plsc.load_gather(amax_v, [kloc])
                plsc.store_scatter(amax_v, [kloc], jnp.maximum(cur, sm),
                                   mask=last & valid)
                return 0
            lax.fori_loop(0, WA // 16, vec, 0)
            return 0
        lax.fori_loop(0, nwin, pass1_win, 0)


        def pass2_win(j, _):
            estart = e0 + j * WA
            stage(estart)

            def vec(v, _):
                base = v * 16
                alpha, valid, key, kloc = alpha_of(base, estart)
                am = plsc.load_gather(amax_v, [kloc])
                ex = jnp.exp(jnp.where(valid, alpha - am, jnp.float32(NEG)))
                plsc.addupdate_scatter(den_v, [kloc], ex, mask=valid)
                return 0
            lax.fori_loop(0, WA // 16, vec, 0)
            return 0
        lax.fori_loop(0, nwin, pass2_win, 0)

        sems = (sem0, sem1)

        def fire(s):
            par = s % 2
            sbase = s * SB
            cps = []
            for h in range(SB // 128):
                idx_ref = s3_v.at[1, pl.ds(sbase + h * 128, 128)]
                dst_ref = rb_v.at[par, pl.ds(h * 128, 128), :]
                cps.append(pltpu.async_copy(xw_h.at[idx_ref], dst_ref,
                                            sems[par]))
            return cps

        def passb_win(j, carry):
            estart = e0 + j * WA
            stage(estart)
            carry_in = carry
            cps = fire(0)
            for s in range(NSB):
                par = s % 2
                sbase = s * SB
                nxt = fire(s + 1) if s + 1 < NSB else None
                def avec(v, _):
                    base = sbase + v * 16
                    alpha, valid, key, kloc = alpha_of(base, estart)
                    am = plsc.load_gather(amax_v, [kloc])
                    den = plsc.load_gather(den_v, [kloc])
                    ex = jnp.exp(jnp.where(valid, alpha - am,
                                           jnp.float32(NEG)))
                    a = ex / (den + jnp.float32(1e-16))
                    a_v[pl.ds(v * 16, 16)] = jnp.where(valid, a,
                                                       jnp.float32(0.0))
                    return 0
                lax.fori_loop(0, SB // 16, avec, 0)
                for cp in cps:
                    cp.wait()

                def edgegrp(g, car):
                    acc0, acc1, acc2, acc3, prev = car
                    accs = [acc0, acc1, acc2, acc3]
                    a16 = a_v[pl.ds(g * 16, 16)]
                    d16 = s3_v[2, pl.ds(sbase + g * 16, 16)]
                    kbuf[pl.ds(16, 16)] = d16
                    sh = kbuf[pl.ds(15, 16)]
                    samef = jnp.where(d16 == sh, jnp.float32(1.0),
                                      jnp.float32(0.0))
                    dl16 = d16 - node_lo
                    ok16 = (dl16 >= 0) & (dl16 < NPS)
                    off16 = jnp.where(ok16, dl16, DUMP) * D_H
                    same0 = jnp.where(d16[0] == prev, jnp.float32(1.0),
                                      jnp.float32(0.0))
                    for lane in range(16):
                        ai = a16[lane]
                        sf = same0 if lane == 0 else samef[lane]
                        off = off16[lane]
                        for c in range(D_H // 16):
                            acc = sf * accs[c] + ai * rb_v[
                                par, g * 16 + lane, pl.ds(c * 16, 16)]
                            out_v[pl.ds(off + c * 16, 16)] = acc
                            accs[c] = acc
                    return accs[0], accs[1], accs[2], accs[3], d16[15]
                carry_in = lax.fori_loop(0, SB // 16, edgegrp, carry_in)
                cps = nxt
            return carry_in

        carry0 = (zeros16, zeros16, zeros16, zeros16, jnp.int32(-1))
        lax.fori_loop(0, nwin, passb_win, carry0)

        pltpu.sync_copy(
            out_v.at[pl.ds(0, NPS * D_H)],
            msg_h.at[pl.ds(pl.multiple_of(node_lo * D_H, 8), NPS * D_H)])

    return sck(xw_flat, qn_flat, kn_flat, e3, ebnd)


def _tc_aggregate(nr, bix, t):
    DT = NLAYERS * D_H
    t2 = jnp.reshape(t, (1, 1))

    def body1(nr_ref, b_ref, t_ref, am_ref):
        @pl.when(pl.program_id(0) == 0)
        def _():
            am_ref[...] = jnp.full((G, DT), NEG, _f32)
        xb = nr_ref[...] * t_ref[0, 0]
        b = b_ref[...]
        for g in range(G):
            m = jnp.max(jnp.where(b == g, xb, jnp.float32(NEG)),
                        axis=0, keepdims=True)
            am_ref[pl.ds(g, 1), :] = jnp.maximum(am_ref[pl.ds(g, 1), :], m)

    amax = pl.pallas_call(
        body1, grid=(NT // BN,),
        in_specs=[pl.BlockSpec((BN, DT), lambda i: (i, 0)),
                  pl.BlockSpec((BN, 1), lambda i: (i, 0)),
                  pl.BlockSpec((1, 1), lambda i: (0, 0))],
        out_specs=pl.BlockSpec((G, DT), lambda i: (0, 0)),
        out_shape=jax.ShapeDtypeStruct((G, DT), _f32),
    )(nr, bix, t2)

    def body2(nr_ref, b_ref, t_ref, am_ref, o_ref, sex_ref, sxex_ref):
        i = pl.program_id(0)

        @pl.when(i == 0)
        def _():
            sex_ref[...] = jnp.zeros((G, DT), _f32)
            sxex_ref[...] = jnp.zeros((G, DT), _f32)
        x = nr_ref[...]
        xb = x * t_ref[0, 0]
        b = b_ref[...]
        oh = (b == lax.broadcasted_iota(_i32, (BN, G), 1)).astype(_f32)
        am_rows = jnp.dot(oh, am_ref[...], preferred_element_type=_f32,
                          precision=lax.Precision.HIGHEST)
        ex = jnp.exp(xb - am_rows)
        dn = (((0,), (0,)), ((), ()))
        sex_ref[...] += lax.dot_general(
            oh, ex, dn, preferred_element_type=_f32,
            precision=lax.Precision.HIGHEST)
        sxex_ref[...] += lax.dot_general(
            oh, x * ex, dn, preferred_element_type=_f32,
            precision=lax.Precision.HIGHEST)

        @pl.when(i == NT // BN - 1)
        def _():
            o_ref[...] = sxex_ref[...] / (sex_ref[...] + jnp.float32(1e-16))

    return pl.pallas_call(
        body2, grid=(NT // BN,),
        in_specs=[pl.BlockSpec((BN, DT), lambda i: (i, 0)),
                  pl.BlockSpec((BN, 1), lambda i: (i, 0)),
                  pl.BlockSpec((1, 1), lambda i: (0, 0)),
                  pl.BlockSpec((G, DT), lambda i: (0, 0))],
        out_specs=pl.BlockSpec((G, DT), lambda i: (0, 0)),
        out_shape=jax.ShapeDtypeStruct((G, DT), _f32),
        scratch_shapes=[pltpu.VMEM((G, DT), _f32),
                        pltpu.VMEM((G, DT), _f32)],
    )(nr, bix, t2, amax)


def kernel(node_features, edge_index, edge_type, batch_index,
           W0, Ws, q_att, k_att, biases, t):
    src = edge_index[0]
    dst = edge_index[1]
    perm = jnp.argsort(dst)
    dst_s = dst[perm]
    src_s = src[perm]
    et_s = edge_type[perm]
    qidx = et_s * NT + dst_s
    kidx = et_s * NT + src_s
    padi = jnp.zeros((EPAD - E,), _i32)
    qidx = jnp.concatenate([qidx, padi])
    kidx = jnp.concatenate([kidx, padi])
    dst_p = jnp.concatenate([dst_s, jnp.full((EPAD - E,), 1 << 28, _i32)])
    e3 = jnp.stack([qidx, kidx, dst_p])
    ebnd = jnp.searchsorted(
        dst_s, jnp.minimum(jnp.arange(33, dtype=_i32) * NPS, N)).astype(_i32)
    ebnd = jnp.concatenate([ebnd, jnp.full((31,), E, _i32)])

    x0 = jnp.concatenate(
        [node_features, jnp.zeros((NT - N, node_features.shape[1]), _f32)])
    bix = jnp.concatenate(
        [batch_index, jnp.full((NT - N,), G, _i32)]).reshape(NT, 1)

    results = []
    xw, qn, kn, _ = _tc_layer(x0, W0, q_att[0], k_att[0], None)
    msg = _sc_layer(xw.reshape(R * NT, D_H), qn.reshape(-1), kn.reshape(-1),
                    e3, ebnd).reshape(NT, D_H)
    for i in range(NLAYERS - 1):
        xw, qn, kn, h = _tc_layer(msg, Ws[i], q_att[i + 1], k_att[i + 1],
                                  biases[i])
        results.append(h)
        msg = _sc_layer(xw.reshape(R * NT, D_H), qn.reshape(-1),
                        kn.reshape(-1), e3, ebnd).reshape(NT, D_H)
    results.append(_tc_bias(msg, biases[NLAYERS - 1]))
    nr_pad = jnp.concatenate(results, axis=-1)
    graph_representations = _tc_aggregate(nr_pad, bix, t)
    node_representations = nr_pad[:N]
    return (graph_representations, node_representations)
